# Initial kernel scaffold; baseline (speedup 1.0000x reference)
import jax
import jax.numpy as jnp
from jax.experimental import pallas as pl


def kernel(x, edge_index, W1, b1, Wg, att_src, att_dst, bg, W2, b2, W3, b3, Wl, bl):
    raise NotImplementedError("write your pallas kernel here")


# speedup vs baseline: 1.0128x; 1.0128x over previous
import jax
import jax.numpy as jnp
from jax.experimental import pallas as pl

N = 10000
E = 320000
F_IN = 128
HID = 64
OUT = 64
HEADS = 12


def _mm_kernel(x_ref, w_ref, o_ref):
    o_ref[...] = jnp.dot(x_ref[...], w_ref[...], preferred_element_type=jnp.float32)


def _mm(x, w):
    return pl.pallas_call(
        _mm_kernel,
        out_shape=jax.ShapeDtypeStruct((x.shape[0], w.shape[1]), jnp.float32),
    )(x, w)


def _gcn(x, src, dst, W, b, s_all, d_all, dinv):
    h = _mm(x, W)
    norm = dinv[s_all] * dinv[d_all]
    msg = h[s_all] * norm[:, None]
    out = jax.ops.segment_sum(msg, d_all, num_segments=N)
    return out + b


def kernel(x, edge_index, W1, b1, Wg, att_src, att_dst, bg, W2, b2, W3, b3, Wl, bl):
    src = edge_index[0]
    dst = edge_index[1]
    loop = jnp.arange(N)
    s_all = jnp.concatenate([src, loop])
    d_all = jnp.concatenate([dst, loop])
    deg = jax.ops.segment_sum(jnp.ones_like(d_all, dtype=jnp.float32), d_all, num_segments=N)
    dinv = jnp.where(deg > 0, deg ** -0.5, 0.0)

    x1 = jax.nn.relu(_gcn(x, src, dst, W1, b1, s_all, d_all, dinv))

    h = _mm(x1, Wg).reshape(N, HEADS, HID)
    a_src = jnp.sum(h * att_src[None], axis=-1)
    a_dst = jnp.sum(h * att_dst[None], axis=-1)
    e = jax.nn.leaky_relu(a_src[s_all] + a_dst[d_all], 0.2)
    emax = jax.ops.segment_max(e, d_all, num_segments=N)
    emax = jnp.where(jnp.isfinite(emax), emax, 0.0)
    ex = jnp.exp(e - emax[d_all])
    denom = jax.ops.segment_sum(ex, d_all, num_segments=N)
    alpha = ex / (denom[d_all] + 1e-16)
    msg = h[s_all] * alpha[:, :, None]
    gat = jax.ops.segment_sum(msg, d_all, num_segments=N).reshape(N, HEADS * HID) + bg
    x2 = jax.nn.elu(gat)

    x3 = jax.nn.relu(_gcn(x2, src, dst, W2, b2, s_all, d_all, dinv))
    x4 = jax.nn.relu(_gcn(x3, src, dst, W3, b3, s_all, d_all, dinv)) + x3
    x4 = jnp.mean(x4, axis=0, keepdims=True)
    return _mm(x4, Wl) + bl


# speedup vs baseline: 1.2371x; 1.2215x over previous
import functools

import jax
import jax.numpy as jnp
from jax import lax
from jax.experimental import pallas as pl
from jax.experimental.pallas import tpu as pltpu
from jax.experimental.pallas import tpu_sc as plsc

N = 10000
E = 320000
F_IN = 128
HID = 64
OUT = 64
HEADS = 12

NC = 2
NS = 16
NW = NC * NS
CHUNK = 80
EPW = E // NW
NCHUNK = EPW // CHUNK
RING = 5
NP = 10240
RPS = NP // NS

_MESH = plsc.VectorSubcoreMesh(core_axis_name="c", subcore_axis_name="s")
_SC_PARAMS = pltpu.CompilerParams(use_tc_tiling_on_sc=False)


def _mm_kernel(x_ref, w_ref, o_ref):
    o_ref[...] = jnp.dot(x_ref[...], w_ref[...], preferred_element_type=jnp.float32)


def _mm(x, w):
    return pl.pallas_call(
        _mm_kernel,
        out_shape=jax.ShapeDtypeStruct((x.shape[0], w.shape[1]), jnp.float32),
    )(x, w)


def _agg_sc(source, sidx, didx, zeros):
    W = source.shape[1]

    @functools.partial(
        pl.kernel,
        out_type=jax.ShapeDtypeStruct((NC, NP, W), jnp.float32),
        mesh=_MESH,
        compiler_params=_SC_PARAMS,
        scratch_types=[
            pltpu.VMEM((NCHUNK, CHUNK), jnp.int32),
            pltpu.VMEM((NCHUNK, CHUNK), jnp.int32),
            [pltpu.VMEM((CHUNK, W), jnp.float32) for _ in range(RING)],
            pltpu.VMEM_SHARED((NP, W), jnp.float32),
            [pltpu.SemaphoreType.DMA for _ in range(RING)],
            pltpu.SemaphoreType.DMA,
        ],
    )
    def k(src_hbm, sidx_hbm, didx_hbm, zero_hbm, out_hbm,
          sidx_v, didx_v, rows, acc, gsems, sem):
        cid = lax.axis_index("c")
        sid = lax.axis_index("s")
        wid = sid * NC + cid
        base = sid * RPS
        pltpu.sync_copy(zero_hbm, acc.at[pl.ds(base, RPS)])
        pltpu.sync_copy(sidx_hbm.at[wid], sidx_v)
        pltpu.sync_copy(didx_hbm.at[wid], didx_v)
        plsc.subcore_barrier()
        for b in range(RING):
            pltpu.async_copy(src_hbm.at[sidx_v.at[b]], rows[b], gsems[b])

        @pl.loop(0, NCHUNK, step=RING)
        def _(j):
            for b in range(RING):
                ch = j + b
                pltpu.make_async_copy(
                    src_hbm.at[sidx_v.at[ch]], rows[b], gsems[b]).wait()
                pltpu.sync_copy(rows[b], acc.at[didx_v.at[ch]], add=True)

                @pl.when(ch + RING < NCHUNK)
                def _():
                    pltpu.async_copy(
                        src_hbm.at[sidx_v.at[ch + RING]], rows[b], gsems[b])

        plsc.subcore_barrier()
        pltpu.sync_copy(acc.at[pl.ds(base, RPS)],
                        out_hbm.at[cid, pl.ds(base, RPS)])

    return k(source, sidx, didx, zeros)


def _deg_sc(didx, ones, zeros):
    @functools.partial(
        pl.kernel,
        out_type=jax.ShapeDtypeStruct((NC, NP, 16), jnp.float32),
        mesh=_MESH,
        compiler_params=_SC_PARAMS,
        scratch_types=[
            pltpu.VMEM((NCHUNK, CHUNK), jnp.int32),
            pltpu.VMEM((CHUNK, 16), jnp.float32),
            pltpu.VMEM_SHARED((NP, 16), jnp.float32),
            pltpu.SemaphoreType.DMA,
        ],
    )
    def k(didx_hbm, ones_hbm, zero_hbm, out_hbm, didx_v, rows, acc, sem):
        cid = lax.axis_index("c")
        sid = lax.axis_index("s")
        wid = sid * NC + cid
        base = sid * RPS
        pltpu.sync_copy(zero_hbm, acc.at[pl.ds(base, RPS)])
        pltpu.sync_copy(didx_hbm.at[wid], didx_v)
        pltpu.sync_copy(ones_hbm, rows)
        plsc.subcore_barrier()

        @pl.loop(0, NCHUNK, step=25)
        def _(j):
            @pl.loop(0, 25)
            def _(b):
                pltpu.async_copy(rows, acc.at[didx_v.at[j + b]], sem, add=True)

            @pl.loop(0, 25)
            def _(b):
                pltpu.make_async_copy(rows, acc.at[didx_v.at[j + b]], sem).wait()

        plsc.subcore_barrier()
        pltpu.sync_copy(acc.at[pl.ds(base, RPS)],
                        out_hbm.at[cid, pl.ds(base, RPS)])

    return k(didx, ones, zeros)


def kernel(x, edge_index, W1, b1, Wg, att_src, att_dst, bg, W2, b2, W3, b3, Wl, bl):
    src = edge_index[0]
    dst = edge_index[1]
    sidx = src.reshape(NW, NCHUNK, CHUNK)
    didx = dst.reshape(NW, NCHUNK, CHUNK)
    z64 = jnp.zeros((RPS, HID), jnp.float32)
    z16 = jnp.zeros((RPS, 16), jnp.float32)
    ones16 = jnp.ones((CHUNK, 16), jnp.float32)

    degp = _deg_sc(didx, ones16, z16)
    deg = degp[0, :N, 0] + degp[1, :N, 0] + 1.0
    dinv = deg ** -0.5

    def gcn(xin, W, b):
        h = _mm(xin, W)
        g = dinv[:, None] * h
        p = _agg_sc(g, sidx, didx, z64)
        return dinv[:, None] * (p[0, :N] + p[1, :N]) + dinv[:, None] ** 2 * h + b

    x1 = jax.nn.relu(gcn(x, W1, b1))

    loop = jnp.arange(N)
    s_all = jnp.concatenate([src, loop])
    d_all = jnp.concatenate([dst, loop])
    h = _mm(x1, Wg).reshape(N, HEADS, HID)
    a_src = jnp.sum(h * att_src[None], axis=-1)
    a_dst = jnp.sum(h * att_dst[None], axis=-1)
    e = jax.nn.leaky_relu(a_src[s_all] + a_dst[d_all], 0.2)
    emax = jax.ops.segment_max(e, d_all, num_segments=N)
    emax = jnp.where(jnp.isfinite(emax), emax, 0.0)
    ex = jnp.exp(e - emax[d_all])
    denom = jax.ops.segment_sum(ex, d_all, num_segments=N)
    alpha = ex / (denom[d_all] + 1e-16)
    msg = h[s_all] * alpha[:, :, None]
    gat = jax.ops.segment_sum(msg, d_all, num_segments=N).reshape(N, HEADS * HID) + bg
    x2 = jax.nn.elu(gat)

    x3 = jax.nn.relu(gcn(x2, W2, b2))
    x4 = jax.nn.relu(gcn(x3, W3, b3)) + x3
    x4 = jnp.mean(x4, axis=0, keepdims=True)
    return _mm(x4, Wl) + bl


# speedup vs baseline: 24.2955x; 19.6396x over previous
import functools

import jax
import jax.numpy as jnp
from jax import lax
from jax.experimental import pallas as pl
from jax.experimental.pallas import tpu as pltpu
from jax.experimental.pallas import tpu_sc as plsc

N = 10000
E = 320000
F_IN = 128
HID = 64
OUT = 64
HEADS = 12

NC = 2
NS = 16
NW = NC * NS
CHUNK = 80
EPW = E // NW
NCHUNK = EPW // CHUNK
RING = 5
NP = 10240
RPS = NP // NS

_MESH = plsc.VectorSubcoreMesh(core_axis_name="c", subcore_axis_name="s")
_SC_PARAMS = pltpu.CompilerParams(use_tc_tiling_on_sc=False)
_SC_PARAMS_NL = pltpu.CompilerParams(use_tc_tiling_on_sc=False,
                                     needs_layout_passes=False)


def _mm_kernel(x_ref, w_ref, o_ref):
    o_ref[...] = jnp.dot(x_ref[...], w_ref[...], preferred_element_type=jnp.float32)


def _mm(x, w):
    return pl.pallas_call(
        _mm_kernel,
        out_shape=jax.ShapeDtypeStruct((x.shape[0], w.shape[1]), jnp.float32),
    )(x, w)


def _agg_sc(source, sidx, didx, zeros):
    W = source.shape[1]

    @functools.partial(
        pl.kernel,
        out_type=jax.ShapeDtypeStruct((NC, NP, W), jnp.float32),
        mesh=_MESH,
        compiler_params=_SC_PARAMS,
        scratch_types=[
            pltpu.VMEM((NCHUNK, CHUNK), jnp.int32),
            pltpu.VMEM((NCHUNK, CHUNK), jnp.int32),
            [pltpu.VMEM((CHUNK, W), jnp.float32) for _ in range(RING)],
            pltpu.VMEM_SHARED((NP, W), jnp.float32),
            [pltpu.SemaphoreType.DMA for _ in range(RING)],
            pltpu.SemaphoreType.DMA,
        ],
    )
    def k(src_hbm, sidx_hbm, didx_hbm, zero_hbm, out_hbm,
          sidx_v, didx_v, rows, acc, gsems, sem):
        cid = lax.axis_index("c")
        sid = lax.axis_index("s")
        wid = sid * NC + cid
        base = sid * RPS
        pltpu.sync_copy(zero_hbm, acc.at[pl.ds(base, RPS)])
        pltpu.sync_copy(sidx_hbm.at[wid], sidx_v)
        pltpu.sync_copy(didx_hbm.at[wid], didx_v)
        plsc.subcore_barrier()
        for b in range(RING):
            pltpu.async_copy(src_hbm.at[sidx_v.at[b]], rows[b], gsems[b])

        @pl.loop(0, NCHUNK, step=RING)
        def _(j):
            for b in range(RING):
                ch = j + b
                pltpu.make_async_copy(
                    src_hbm.at[sidx_v.at[ch]], rows[b], gsems[b]).wait()
                pltpu.sync_copy(rows[b], acc.at[didx_v.at[ch]], add=True)

                @pl.when(ch + RING < NCHUNK)
                def _():
                    pltpu.async_copy(
                        src_hbm.at[sidx_v.at[ch + RING]], rows[b], gsems[b])

        plsc.subcore_barrier()
        pltpu.sync_copy(acc.at[pl.ds(base, RPS)],
                        out_hbm.at[cid, pl.ds(base, RPS)])

    return k(source, sidx, didx, zeros)


def _deg_sc(didx, ones, zeros):
    @functools.partial(
        pl.kernel,
        out_type=jax.ShapeDtypeStruct((NC, NP, 16), jnp.float32),
        mesh=_MESH,
        compiler_params=_SC_PARAMS,
        scratch_types=[
            pltpu.VMEM((NCHUNK, CHUNK), jnp.int32),
            pltpu.VMEM((CHUNK, 16), jnp.float32),
            pltpu.VMEM_SHARED((NP, 16), jnp.float32),
            pltpu.SemaphoreType.DMA,
        ],
    )
    def k(didx_hbm, ones_hbm, zero_hbm, out_hbm, didx_v, rows, acc, sem):
        cid = lax.axis_index("c")
        sid = lax.axis_index("s")
        wid = sid * NC + cid
        base = sid * RPS
        pltpu.sync_copy(zero_hbm, acc.at[pl.ds(base, RPS)])
        pltpu.sync_copy(didx_hbm.at[wid], didx_v)
        pltpu.sync_copy(ones_hbm, rows)
        plsc.subcore_barrier()

        @pl.loop(0, NCHUNK, step=25)
        def _(j):
            @pl.loop(0, 25)
            def _(b):
                pltpu.async_copy(rows, acc.at[didx_v.at[j + b]], sem, add=True)

            @pl.loop(0, 25)
            def _(b):
                pltpu.make_async_copy(rows, acc.at[didx_v.at[j + b]], sem).wait()

        plsc.subcore_barrier()
        pltpu.sync_copy(acc.at[pl.ds(base, RPS)],
                        out_hbm.at[cid, pl.ds(base, RPS)])

    return k(didx, ones, zeros)


N2 = 2 * N
EPS = E // NS
NCH2 = EPS // CHUNK
GRP = 25
NGRP = NCH2 // GRP
HPC = HEADS // NC
RPH = N2 // NS
ZR = 50


def _mask_sc(asrcT, adstT, sidx, didx):
    @functools.partial(
        pl.kernel,
        out_type=jax.ShapeDtypeStruct((NW, NCHUNK, CHUNK), jnp.int32),
        mesh=_MESH,
        compiler_params=_SC_PARAMS_NL,
        scratch_types=[
            pltpu.VMEM((NCHUNK, CHUNK), jnp.int32),
            pltpu.VMEM((NCHUNK, CHUNK), jnp.int32),
            pltpu.VMEM((NCHUNK, CHUNK), jnp.int32),
            pltpu.VMEM((N,), jnp.float32),
            pltpu.VMEM((N,), jnp.float32),
        ],
    )
    def k(asrc_hbm, adst_hbm, sidx_hbm, didx_hbm, out_hbm,
          sidx_v, didx_v, mb_v, ta, tb):
        cid = lax.axis_index("c")
        sid = lax.axis_index("s")
        wid = sid * NC + cid
        pltpu.sync_copy(sidx_hbm.at[wid], sidx_v)
        pltpu.sync_copy(didx_hbm.at[wid], didx_v)

        @pl.loop(0, HEADS)
        def _(h):
            pltpu.sync_copy(asrc_hbm.at[h], ta)
            pltpu.sync_copy(adst_hbm.at[h], tb)

            @pl.loop(0, NCHUNK)
            def _(ch):
                for kq in range(CHUNK // 16):
                    sl = pl.ds(kq * 16, 16)
                    s16 = sidx_v[ch, sl]
                    d16 = didx_v[ch, sl]
                    av = plsc.load_gather(ta, [s16])
                    bv = plsc.load_gather(tb, [d16])
                    m = jnp.where(av + bv < 0.0, jnp.int32(1), jnp.int32(0))
                    mb_v[ch, sl] = jnp.where(
                        h == 0, m, mb_v[ch, sl] | (m << h))

        pltpu.sync_copy(mb_v, out_hbm.at[wid])

    return k(asrcT, adstT, sidx, didx)


def _gat_sc(srcrows, sidx2, didx2, mbits2, zeros):
    @functools.partial(
        pl.kernel,
        out_type=jax.ShapeDtypeStruct((HEADS, N2, 80), jnp.float32),
        mesh=_MESH,
        compiler_params=_SC_PARAMS_NL,
        scratch_types=[
            [[pltpu.VMEM((GRP, CHUNK), jnp.int32) for _ in range(3)]
             for _ in range(2)],
            [pltpu.VMEM((1, CHUNK), jnp.int32) for _ in range(2)],
            [pltpu.VMEM((1, CHUNK), jnp.int32) for _ in range(2)],
            [pltpu.VMEM((CHUNK, 80), jnp.float32) for _ in range(2)],
            pltpu.VMEM((ZR, 80), jnp.float32),
            pltpu.VMEM_SHARED((N2, 80), jnp.float32),
            [pltpu.SemaphoreType.DMA for _ in range(2)],
            [pltpu.SemaphoreType.DMA for _ in range(2)],
        ],
    )
    def k(src_hbm, sidx_hbm, didx_hbm, mb_hbm, zero_hbm, out_hbm,
          grp, gidx, scidx, rows, zbuf, acc, gsems, isems):
        cid = lax.axis_index("c")
        sid = lax.axis_index("s")
        base = sid * RPH
        pltpu.sync_copy(zero_hbm, zbuf)

        def fetch_group(g, par):
            pltpu.async_copy(sidx_hbm.at[sid, pl.ds(g * GRP, GRP)],
                             grp[par][0], isems[par])
            pltpu.async_copy(didx_hbm.at[sid, pl.ds(g * GRP, GRP)],
                             grp[par][1], isems[par])
            pltpu.async_copy(mb_hbm.at[sid, pl.ds(g * GRP, GRP)],
                             grp[par][2], isems[par])

        def wait_group(g, par):
            for q in range(3):
                pltpu.make_async_copy(
                    sidx_hbm.at[sid, pl.ds(g * GRP, GRP)],
                    grp[par][q], isems[par]).wait()

        def compute_idx(h, par, lc, b):
            head_base = h * N2
            for kq in range(CHUNK // 16):
                sl = pl.ds(kq * 16, 16)
                s16 = grp[par][0][lc, sl]
                d16 = grp[par][1][lc, sl]
                mb = grp[par][2][lc, sl]
                off = jnp.where((mb >> h) & 1 == 1,
                                jnp.int32(N), jnp.int32(0))
                gidx[b][0, sl] = s16 + off + head_base
                scidx[b][0, sl] = d16 + off

        def proc(issue_next, h, par, lc, b, nlc=0):
            pltpu.make_async_copy(
                src_hbm.at[gidx[b].at[0]], rows[b], gsems[b]).wait()
            pltpu.sync_copy(rows[b], acc.at[scidx[b].at[0]], add=True)
            if issue_next:
                compute_idx(h, par, nlc, b)
                pltpu.async_copy(src_hbm.at[gidx[b].at[0]], rows[b], gsems[b])

        fetch_group(0, 0)

        @pl.loop(0, HPC)
        def _(hl):
            h = cid * HPC + hl
            for z in range(RPH // ZR):
                pltpu.sync_copy(zbuf, acc.at[pl.ds(base + z * ZR, ZR)])
            plsc.subcore_barrier()

            @pl.loop(0, NGRP, step=2)
            def _(g):
                for par in range(2):
                    ge = g + par
                    wait_group(ge, par)
                    fetch_group((ge + 1) % NGRP, 1 - par)
                    compute_idx(h, par, 0, 0)
                    pltpu.async_copy(
                        src_hbm.at[gidx[0].at[0]], rows[0], gsems[0])
                    compute_idx(h, par, 1, 1)
                    pltpu.async_copy(
                        src_hbm.at[gidx[1].at[0]], rows[1], gsems[1])

                    @pl.loop(0, GRP - 3, step=2)
                    def _(lc):
                        proc(True, h, par, lc, 0, lc + 2)
                        proc(True, h, par, lc + 1, 1, lc + 3)

                    proc(True, h, par, GRP - 3, 0, GRP - 1)
                    proc(False, h, par, GRP - 2, 1)
                    proc(False, h, par, GRP - 1, 0)

            plsc.subcore_barrier()
            pltpu.sync_copy(acc.at[pl.ds(base, RPH)],
                            out_hbm.at[h, pl.ds(base, RPH)])

        wait_group(0, 0)

    return k(srcrows, sidx2, didx2, mbits2, zeros)


def kernel(x, edge_index, W1, b1, Wg, att_src, att_dst, bg, W2, b2, W3, b3, Wl, bl):
    src = edge_index[0]
    dst = edge_index[1]
    sidx = src.reshape(NW, NCHUNK, CHUNK)
    didx = dst.reshape(NW, NCHUNK, CHUNK)
    z64 = jnp.zeros((RPS, HID), jnp.float32)
    z16 = jnp.zeros((RPS, 16), jnp.float32)
    ones16 = jnp.ones((CHUNK, 16), jnp.float32)

    degp = _deg_sc(didx, ones16, z16)
    deg = degp[0, :N, 0] + degp[1, :N, 0] + 1.0
    dinv = deg ** -0.5

    def gcn(xin, W, b):
        h = _mm(xin, W)
        g = dinv[:, None] * h
        p = _agg_sc(g, sidx, didx, z64)
        return dinv[:, None] * (p[0, :N] + p[1, :N]) + dinv[:, None] ** 2 * h + b

    x1 = jax.nn.relu(gcn(x, W1, b1))

    h = _mm(x1, Wg)
    hh = h.reshape(N, HEADS, HID)
    a_src = jnp.sum(hh * att_src[None], axis=-1)
    a_dst = jnp.sum(hh * att_dst[None], axis=-1)
    gmax = jnp.max(a_src, axis=0)
    zc = a_dst + gmax
    c = jnp.maximum(zc, 0.2 * zc)
    B1 = jnp.exp(zc - c)
    B2 = jnp.exp(0.2 * zc - c)
    A1 = jnp.exp(a_src - gmax)
    A2 = jnp.exp(0.2 * (a_src - gmax))
    hT = hh.transpose(1, 0, 2)
    z15 = jnp.zeros((HEADS, N, 15), jnp.float32)

    def rows_for(A):
        AT = A.T[:, :, None]
        return jnp.concatenate([AT * hT, AT, z15], axis=-1)

    srcrows = jnp.concatenate([rows_for(A1), rows_for(A2)], axis=1)
    srcrows = srcrows.reshape(HEADS * 2 * N, 80)
    z80 = jnp.zeros((ZR, 80), jnp.float32)
    mbits = _mask_sc(a_src.T, a_dst.T, sidx, didx)
    sidx2 = src.reshape(NS, NCH2, CHUNK)
    didx2 = dst.reshape(NS, NCH2, CHUNK)
    mbits2 = mbits.reshape(NS, NCH2, CHUNK)
    P = _gat_sc(srcrows, sidx2, didx2, mbits2, z80)
    u_self = a_src + a_dst
    ex_self = jnp.exp(jnp.maximum(u_self, 0.2 * u_self) - c)
    R1 = P[:, :N, :HID].transpose(1, 0, 2)
    R2 = P[:, N:2 * N, :HID].transpose(1, 0, 2)
    S1 = P[:, :N, HID].T
    S2 = P[:, N:2 * N, HID].T
    num = (B1[:, :, None] * R1 + B2[:, :, None] * R2 + ex_self[:, :, None] * hh)
    den = B1 * S1 + B2 * S2 + ex_self
    gat = (num / (den[:, :, None] + 1e-16)).reshape(N, HEADS * HID) + bg
    x2 = jax.nn.elu(gat)

    x3 = jax.nn.relu(gcn(x2, W2, b2))
    x4 = jax.nn.relu(gcn(x3, W3, b3)) + x3
    x4 = jnp.mean(x4, axis=0, keepdims=True)
    return _mm(x4, Wl) + bl


# speedup vs baseline: 32.3929x; 1.3333x over previous
import functools

import jax
import jax.numpy as jnp
from jax import lax
from jax.experimental import pallas as pl
from jax.experimental.pallas import tpu as pltpu
from jax.experimental.pallas import tpu_sc as plsc

N = 10000
E = 320000
F_IN = 128
HID = 64
OUT = 64
HEADS = 12

NC = 2
NS = 16
NW = NC * NS
CHUNK = 80
EPW = E // NW
NCHUNK = EPW // CHUNK
RING = 5
NP = 10240
RPS = NP // NS

_MESH = plsc.VectorSubcoreMesh(core_axis_name="c", subcore_axis_name="s")
_SC_PARAMS = pltpu.CompilerParams(use_tc_tiling_on_sc=False)
_SC_PARAMS_NL = pltpu.CompilerParams(use_tc_tiling_on_sc=False,
                                     needs_layout_passes=False)


def _mm_kernel(x_ref, w_ref, o_ref):
    o_ref[...] = jnp.dot(x_ref[...], w_ref[...], preferred_element_type=jnp.float32)


def _mm(x, w):
    return pl.pallas_call(
        _mm_kernel,
        out_shape=jax.ShapeDtypeStruct((x.shape[0], w.shape[1]), jnp.float32),
    )(x, w)


def _agg_sc(source, sidx, didx, zeros):
    W = source.shape[1]

    @functools.partial(
        pl.kernel,
        out_type=jax.ShapeDtypeStruct((NC, NP, W), jnp.float32),
        mesh=_MESH,
        compiler_params=_SC_PARAMS,
        scratch_types=[
            pltpu.VMEM((NCHUNK, CHUNK), jnp.int32),
            pltpu.VMEM((NCHUNK, CHUNK), jnp.int32),
            [pltpu.VMEM((CHUNK, W), jnp.float32) for _ in range(RING)],
            pltpu.VMEM_SHARED((NP, W), jnp.float32),
            [pltpu.SemaphoreType.DMA for _ in range(RING)],
            pltpu.SemaphoreType.DMA,
        ],
    )
    def k(src_hbm, sidx_hbm, didx_hbm, zero_hbm, out_hbm,
          sidx_v, didx_v, rows, acc, gsems, sem):
        cid = lax.axis_index("c")
        sid = lax.axis_index("s")
        wid = sid * NC + cid
        base = sid * RPS
        pltpu.sync_copy(zero_hbm, acc.at[pl.ds(base, RPS)])
        pltpu.sync_copy(sidx_hbm.at[wid], sidx_v)
        pltpu.sync_copy(didx_hbm.at[wid], didx_v)
        plsc.subcore_barrier()
        for b in range(RING):
            pltpu.async_copy(src_hbm.at[sidx_v.at[b]], rows[b], gsems[b])

        @pl.loop(0, NCHUNK, step=RING)
        def _(j):
            for b in range(RING):
                ch = j + b
                pltpu.make_async_copy(
                    src_hbm.at[sidx_v.at[ch]], rows[b], gsems[b]).wait()
                pltpu.sync_copy(rows[b], acc.at[didx_v.at[ch]], add=True)

                @pl.when(ch + RING < NCHUNK)
                def _():
                    pltpu.async_copy(
                        src_hbm.at[sidx_v.at[ch + RING]], rows[b], gsems[b])

        plsc.subcore_barrier()
        pltpu.sync_copy(acc.at[pl.ds(base, RPS)],
                        out_hbm.at[cid, pl.ds(base, RPS)])

    return k(source, sidx, didx, zeros)


def _deg_sc(didx, ones, zeros):
    @functools.partial(
        pl.kernel,
        out_type=jax.ShapeDtypeStruct((NC, NP, 16), jnp.float32),
        mesh=_MESH,
        compiler_params=_SC_PARAMS,
        scratch_types=[
            pltpu.VMEM((NCHUNK, CHUNK), jnp.int32),
            pltpu.VMEM((CHUNK, 16), jnp.float32),
            pltpu.VMEM_SHARED((NP, 16), jnp.float32),
            pltpu.SemaphoreType.DMA,
        ],
    )
    def k(didx_hbm, ones_hbm, zero_hbm, out_hbm, didx_v, rows, acc, sem):
        cid = lax.axis_index("c")
        sid = lax.axis_index("s")
        wid = sid * NC + cid
        base = sid * RPS
        pltpu.sync_copy(zero_hbm, acc.at[pl.ds(base, RPS)])
        pltpu.sync_copy(didx_hbm.at[wid], didx_v)
        pltpu.sync_copy(ones_hbm, rows)
        plsc.subcore_barrier()

        @pl.loop(0, NCHUNK, step=25)
        def _(j):
            @pl.loop(0, 25)
            def _(b):
                pltpu.async_copy(rows, acc.at[didx_v.at[j + b]], sem, add=True)

            @pl.loop(0, 25)
            def _(b):
                pltpu.make_async_copy(rows, acc.at[didx_v.at[j + b]], sem).wait()

        plsc.subcore_barrier()
        pltpu.sync_copy(acc.at[pl.ds(base, RPS)],
                        out_hbm.at[cid, pl.ds(base, RPS)])

    return k(didx, ones, zeros)


N2 = 2 * N
EPS = E // NS
NCH2 = EPS // CHUNK
GRP = 25
NGRP = NCH2 // GRP
HPC = HEADS // NC
NP2 = 2 * NP
RPH = NP2 // NS


def _mask_sc(asrcT, adstT, sidx, didx):
    @functools.partial(
        pl.kernel,
        out_type=jax.ShapeDtypeStruct((NW, NCHUNK, CHUNK), jnp.int32),
        mesh=_MESH,
        compiler_params=_SC_PARAMS_NL,
        scratch_types=[
            pltpu.VMEM((NCHUNK, CHUNK), jnp.int32),
            pltpu.VMEM((NCHUNK, CHUNK), jnp.int32),
            pltpu.VMEM((NCHUNK, CHUNK), jnp.int32),
            pltpu.VMEM((N,), jnp.float32),
            pltpu.VMEM((N,), jnp.float32),
        ],
    )
    def k(asrc_hbm, adst_hbm, sidx_hbm, didx_hbm, out_hbm,
          sidx_v, didx_v, mb_v, ta, tb):
        cid = lax.axis_index("c")
        sid = lax.axis_index("s")
        wid = sid * NC + cid
        pltpu.sync_copy(sidx_hbm.at[wid], sidx_v)
        pltpu.sync_copy(didx_hbm.at[wid], didx_v)

        @pl.loop(0, HEADS)
        def _(h):
            pltpu.sync_copy(asrc_hbm.at[h], ta)
            pltpu.sync_copy(adst_hbm.at[h], tb)

            @pl.loop(0, NCHUNK)
            def _(ch):
                for kq in range(CHUNK // 16):
                    sl = pl.ds(kq * 16, 16)
                    s16 = sidx_v[ch, sl]
                    d16 = didx_v[ch, sl]
                    av = plsc.load_gather(ta, [s16])
                    bv = plsc.load_gather(tb, [d16])
                    m = jnp.where(av + bv < 0.0, jnp.int32(1), jnp.int32(0))
                    mb_v[ch, sl] = jnp.where(
                        h == 0, m, mb_v[ch, sl] | (m << h))

        @pl.loop(0, NCHUNK)
        def _(ch):
            for kq in range(CHUNK // 16):
                sl = pl.ds(kq * 16, 16)
                mb_v[ch, sl] = didx_v[ch, sl] | (mb_v[ch, sl] << 14)

        pltpu.sync_copy(mb_v, out_hbm.at[wid])

    return k(asrcT, adstT, sidx, didx)


SRC_HB = NP
SRC_B2 = HEADS * NP


def _gat_sc(srcrows, sidx2, dm2, zeros):
    @functools.partial(
        pl.kernel,
        out_type=jax.ShapeDtypeStruct((HEADS, NP2, 80), jnp.float32),
        mesh=_MESH,
        compiler_params=_SC_PARAMS_NL,
        scratch_types=[
            [[pltpu.VMEM((GRP, CHUNK), jnp.int32) for _ in range(2)]
             for _ in range(2)],
            [pltpu.VMEM((1, CHUNK), jnp.int32) for _ in range(3)],
            [pltpu.VMEM((1, CHUNK), jnp.int32) for _ in range(3)],
            [pltpu.VMEM((CHUNK, 80), jnp.float32) for _ in range(3)],
            pltpu.VMEM_SHARED((NP2, 80), jnp.float32),
            [pltpu.SemaphoreType.DMA for _ in range(3)],
            [pltpu.SemaphoreType.DMA for _ in range(2)],
        ],
    )
    def k(src_hbm, sidx_hbm, dm_hbm, zero_hbm, out_hbm,
          grp, gidx, scidx, rows, acc, gsems, isems):
        cid = lax.axis_index("c")
        sid = lax.axis_index("s")
        base = sid * RPH

        def fetch_group(g, par):
            pltpu.async_copy(sidx_hbm.at[sid, pl.ds(g * GRP, GRP)],
                             grp[par][0], isems[par])
            pltpu.async_copy(dm_hbm.at[sid, pl.ds(g * GRP, GRP)],
                             grp[par][1], isems[par])

        def wait_group(g, par):
            for q in range(2):
                pltpu.make_async_copy(
                    sidx_hbm.at[sid, pl.ds(g * GRP, GRP)],
                    grp[par][q], isems[par]).wait()

        def compute_idx(h, par, lc, b):
            head_base = h * SRC_HB
            for kq in range(CHUNK // 16):
                sl = pl.ds(kq * 16, 16)
                s16 = grp[par][0][lc, sl]
                dm = grp[par][1][lc, sl]
                d16 = dm & jnp.int32(16383)
                mbit = (dm >> (14 + h)) & 1
                gidx[b][0, sl] = s16 + head_base + jnp.where(
                    mbit == 1, jnp.int32(SRC_B2), jnp.int32(0))
                scidx[b][0, sl] = d16 + jnp.where(
                    mbit == 1, jnp.int32(NP), jnp.int32(0))

        def proc(issue_next, h, par, lc, b, nlc=0):
            pltpu.make_async_copy(
                src_hbm.at[gidx[b].at[0]], rows[b], gsems[b]).wait()
            pltpu.sync_copy(rows[b], acc.at[scidx[b].at[0]], add=True)
            if issue_next:
                compute_idx(h, par, nlc, b)
                pltpu.async_copy(src_hbm.at[gidx[b].at[0]], rows[b], gsems[b])

        fetch_group(0, 0)

        @pl.loop(0, HPC)
        def _(hl):
            h = cid * HPC + hl
            pltpu.sync_copy(zero_hbm, acc.at[pl.ds(base, RPH)])
            plsc.subcore_barrier()

            @pl.loop(0, NGRP, step=2)
            def _(g):
                for par in range(2):
                    ge = g + par
                    wait_group(ge, par)
                    fetch_group((ge + 1) % NGRP, 1 - par)
                    for b in range(3):
                        compute_idx(h, par, b, b)
                        pltpu.async_copy(
                            src_hbm.at[gidx[b].at[0]], rows[b], gsems[b])

                    @pl.loop(0, GRP - 4, step=3)
                    def _(lc):
                        proc(True, h, par, lc, 0, lc + 3)
                        proc(True, h, par, lc + 1, 1, lc + 4)
                        proc(True, h, par, lc + 2, 2, lc + 5)

                    proc(True, h, par, GRP - 4, 0, GRP - 1)
                    proc(False, h, par, GRP - 3, 1)
                    proc(False, h, par, GRP - 2, 2)
                    proc(False, h, par, GRP - 1, 0)

            plsc.subcore_barrier()
            pltpu.sync_copy(acc.at[pl.ds(base, RPH)],
                            out_hbm.at[h, pl.ds(base, RPH)])

        wait_group(0, 0)

    return k(srcrows, sidx2, dm2, zeros)


NB = 1280


def _build_kernel(h_ref, a_ref, o_ref):
    z = jnp.zeros((NB, 15), jnp.float32)
    hb = h_ref[...]
    for j in range(2):
        a = a_ref[0, 0, j, :]
        scaled = hb[:, j * HID:(j + 1) * HID] * a[:, None]
        o_ref[0, j] = jnp.concatenate([scaled, a[:, None], z], axis=1)


def _build_rows(hpad, Apad):
    return pl.pallas_call(
        _build_kernel,
        grid=(2, HEADS // 2, NP // NB),
        in_specs=[
            pl.BlockSpec((NB, 2 * HID), lambda m, hp, nb: (nb, hp)),
            pl.BlockSpec((1, 1, 2, NB), lambda m, hp, nb: (m, hp, 0, nb)),
        ],
        out_specs=pl.BlockSpec((1, 2, NB, 80), lambda m, hp, nb: (m, hp, nb, 0)),
        out_shape=jax.ShapeDtypeStruct((2, HEADS, NP, 80), jnp.float32),
    )(hpad, Apad.reshape(2, HEADS // 2, 2, NP))


def _post_kernel(p_ref, h_ref, b1_ref, b2_ref, es_ref, bg_ref, o_ref):
    for j in range(2):
        r1 = p_ref[0, j, 0, :, :HID]
        r2 = p_ref[0, j, 1, :, :HID]
        s1 = p_ref[0, j, 0, :, HID]
        s2 = p_ref[0, j, 1, :, HID]
        hb = h_ref[:, j * HID:(j + 1) * HID]
        b1 = b1_ref[0, j, :]
        b2 = b2_ref[0, j, :]
        es = es_ref[0, j, :]
        num = b1[:, None] * r1 + b2[:, None] * r2 + es[:, None] * hb
        den = b1 * s1 + b2 * s2 + es
        v = num / (den[:, None] + 1e-16) + bg_ref[0, j]
        o_ref[:, j * HID:(j + 1) * HID] = jnp.where(v > 0, v, jnp.exp(v) - 1.0)


def _gat_post(P, hpad, B1p, B2p, exsp, bgr):
    return pl.pallas_call(
        _post_kernel,
        grid=(HEADS // 2, NP // NB),
        in_specs=[
            pl.BlockSpec((1, 2, 2, NB, 80), lambda i, nb: (i, 0, 0, nb, 0)),
            pl.BlockSpec((NB, 2 * HID), lambda i, nb: (nb, i)),
            pl.BlockSpec((1, 2, NB), lambda i, nb: (i, 0, nb)),
            pl.BlockSpec((1, 2, NB), lambda i, nb: (i, 0, nb)),
            pl.BlockSpec((1, 2, NB), lambda i, nb: (i, 0, nb)),
            pl.BlockSpec((1, 2, HID), lambda i, nb: (i, 0, 0)),
        ],
        out_specs=pl.BlockSpec((NB, 2 * HID), lambda i, nb: (nb, i)),
        out_shape=jax.ShapeDtypeStruct((NP, HEADS * HID), jnp.float32),
    )(P.reshape(HEADS // 2, 2, 2, NP, 80), hpad, B1p, B2p, exsp, bgr)


def kernel(x, edge_index, W1, b1, Wg, att_src, att_dst, bg, W2, b2, W3, b3, Wl, bl):
    src = edge_index[0]
    dst = edge_index[1]
    sidx = src.reshape(NW, NCHUNK, CHUNK)
    didx = dst.reshape(NW, NCHUNK, CHUNK)
    z64 = jnp.zeros((RPS, HID), jnp.float32)
    z16 = jnp.zeros((RPS, 16), jnp.float32)
    ones16 = jnp.ones((CHUNK, 16), jnp.float32)

    degp = _deg_sc(didx, ones16, z16)
    deg = degp[0, :N, 0] + degp[1, :N, 0] + 1.0
    dinv = deg ** -0.5

    def gcn(xin, W, b):
        h = _mm(xin, W)
        g = dinv[:, None] * h
        p = _agg_sc(g, sidx, didx, z64)
        return dinv[:, None] * (p[0, :N] + p[1, :N]) + dinv[:, None] ** 2 * h + b

    x1 = jax.nn.relu(gcn(x, W1, b1))

    x1p = jnp.pad(x1, ((0, NP - N), (0, 0)))
    h = _mm(x1p, Wg)
    hh = h[:N].reshape(N, HEADS, HID)
    a_src = jnp.sum(hh * att_src[None], axis=-1)
    a_dst = jnp.sum(hh * att_dst[None], axis=-1)
    gmax = jnp.max(a_src, axis=0)
    zc = a_dst + gmax
    c = jnp.maximum(zc, 0.2 * zc)
    B1 = jnp.exp(zc - c)
    B2 = jnp.exp(0.2 * zc - c)
    A1 = jnp.exp(a_src - gmax)
    A2 = jnp.exp(0.2 * (a_src - gmax))
    Apad = jnp.pad(jnp.stack([A1.T, A2.T]), ((0, 0), (0, 0), (0, NP - N)))
    srcrows = _build_rows(h, Apad).reshape(2 * HEADS * NP, 80)
    z80 = jnp.zeros((RPH, 80), jnp.float32)
    dm = _mask_sc(a_src.T, a_dst.T, sidx, didx)
    sidx2 = src.reshape(NS, NCH2, CHUNK)
    dm2 = dm.reshape(NS, NCH2, CHUNK)
    P = _gat_sc(srcrows, sidx2, dm2, z80)
    u_self = a_src + a_dst
    ex_self = jnp.exp(jnp.maximum(u_self, 0.2 * u_self) - c)

    def hpadT(v):
        return jnp.pad(v.T, ((0, 0), (0, NP - N))).reshape(HEADS // 2, 2, NP)

    x2 = _gat_post(P, h, hpadT(B1), hpadT(B2), hpadT(ex_self),
                   bg.reshape(HEADS // 2, 2, HID))[:N]

    x3 = jax.nn.relu(gcn(x2, W2, b2))
    x4 = jax.nn.relu(gcn(x3, W3, b3)) + x3
    x4 = jnp.mean(x4, axis=0, keepdims=True)
    return _mm(x4, Wl) + bl
